# Initial kernel scaffold; baseline (speedup 1.0000x reference)
import functools

import jax
import jax.numpy as jnp
from jax import lax
from jax.experimental import pallas as pl
from jax.experimental.pallas import tpu as pltpu
from jax.experimental.pallas import tpu_sc as plsc

OUT_H = 7
OUT_W = 7
SPATIAL_SCALE = 0.125
SAMPLING_RATIO = 2

L = 16


def _full(v, dtype=jnp.int32):
    return jnp.full((L,), v, dtype=dtype)


def _roi_align_sc(x_table, rois_pad, *, N, C, H, W, K, rois_per_worker):
    CCH = C // L

    mesh = plsc.VectorSubcoreMesh(core_axis_name="c", subcore_axis_name="s")

    @functools.partial(
        pl.kernel,
        mesh=mesh,
        out_type=jax.ShapeDtypeStruct((K, OUT_H * OUT_W * C), jnp.float32),
        scratch_types=[
            pltpu.VMEM((L,), jnp.float32),
            pltpu.VMEM((2, L), jnp.int32),
            pltpu.VMEM((2, L), jnp.float32),
            pltpu.VMEM((8, L), jnp.float32),
            pltpu.VMEM((8 * L,), jnp.int32),
            pltpu.VMEM((8 * L, C), jnp.float32),
            pltpu.VMEM((OUT_H * OUT_W * C,), jnp.float32),
            pltpu.SemaphoreType.DMA,
        ],
    )
    def k(table, rois, out, roi_v, yb_v, ay_v, wb_v, idx_v, rows_v, acc_v, sem):
        wid = lax.axis_index("s") * 2 + lax.axis_index("c")

        iota = jnp.arange(L, dtype=jnp.int32)
        iotaf = iota.astype(jnp.float32)

        def roi_body(i, carry):
            r = wid * rois_per_worker + i
            pltpu.sync_copy(rois.at[r], roi_v)

            def bc(lane):
                return plsc.load_gather(roi_v, [_full(lane)])

            b = bc(0).astype(jnp.int32)
            sy1 = bc(1) * SPATIAL_SCALE
            sx1 = bc(2) * SPATIAL_SCALE
            sy2 = bc(3) * SPATIAL_SCALE
            sx2 = bc(4) * SPATIAL_SCALE
            bin_h = jnp.maximum(sy2 - sy1, 1.0) * (1.0 / OUT_H)
            bin_w = jnp.maximum(sx2 - sx1, 1.0) * (1.0 / OUT_W)
            t = iotaf * 0.5 + 0.25

            ys = sy1 + t * bin_h
            vy = jnp.where((ys >= -1.0) & (ys <= float(H)), 0.25, 0.0)
            yc = jnp.clip(ys, 0.0, float(H - 1))
            y0 = yc.astype(jnp.int32)
            fy = yc - y0.astype(jnp.float32)
            y1i = jnp.minimum(y0 + 1, H - 1)
            bbase = b * (H * W)
            yb_v[0] = bbase + y0 * W
            yb_v[1] = bbase + y1i * W
            ay_v[0] = (1.0 - fy) * vy
            ay_v[1] = fy * vy

            xs = sx1 + t * bin_w
            vx = jnp.where((xs >= -1.0) & (xs <= float(W)), 1.0, 0.0)
            xc = jnp.clip(xs, 0.0, float(W - 1))
            x0 = xc.astype(jnp.int32)
            fx = xc - x0.astype(jnp.float32)
            x1i = jnp.minimum(x0 + 1, W - 1)
            ax0 = (1.0 - fx) * vx
            ax1 = fx * vx

            def ph_body(ph, carry2):
                for iy in range(2):
                    sy = 2 * ph + iy
                    yb0 = plsc.load_gather(yb_v, [_full(0), _full(sy)])
                    yb1 = plsc.load_gather(yb_v, [_full(1), _full(sy)])
                    ay0 = plsc.load_gather(ay_v, [_full(0), _full(sy)])
                    ay1 = plsc.load_gather(ay_v, [_full(1), _full(sy)])
                    g = iy * 4
                    idx_v[pl.ds((g + 0) * L, L)] = yb0 + x0
                    idx_v[pl.ds((g + 1) * L, L)] = yb0 + x1i
                    idx_v[pl.ds((g + 2) * L, L)] = yb1 + x0
                    idx_v[pl.ds((g + 3) * L, L)] = yb1 + x1i
                    wb_v[g + 0] = ay0 * ax0
                    wb_v[g + 1] = ay0 * ax1
                    wb_v[g + 2] = ay1 * ax0
                    wb_v[g + 3] = ay1 * ax1

                pltpu.async_copy(table.at[idx_v], rows_v, sem).wait()

                for pw in range(OUT_W):
                    w = [
                        plsc.load_gather(wb_v, [_full(g), _full(2 * pw + jj)])
                        for g in range(8)
                        for jj in range(2)
                    ]
                    base = (ph * OUT_W + pw) * C
                    for ch in range(CCH):
                        acc = w[0] * rows_v[0, pl.ds(ch * L, L)] * 0.0
                        for g in range(8):
                            for jj in range(2):
                                acc = acc + w[g * 2 + jj] * rows_v[
                                    g * L + 2 * pw + jj, pl.ds(ch * L, L)
                                ]
                        acc_v[pl.ds(base + ch * L, L)] = acc
                return carry2

            lax.fori_loop(0, OUT_H, ph_body, 0)
            pltpu.sync_copy(acc_v, out.at[r])
            return carry

        lax.fori_loop(0, rois_per_worker, roi_body, 0)

    return k(x_table, rois_pad)


def kernel(input, rois):
    N, C, H, W = input.shape
    K = rois.shape[0]
    x_table = jnp.transpose(input, (0, 2, 3, 1)).reshape(N * H * W, C)
    rois_pad = jnp.pad(rois, ((0, 0), (0, L - rois.shape[1])))
    out = _roi_align_sc(
        x_table, rois_pad, N=N, C=C, H=H, W=W, K=K, rois_per_worker=K // 32
    )
    out = out.reshape(K, OUT_H, OUT_W, C)
    return jnp.transpose(out, (0, 3, 1, 2))


# speedup vs baseline: 14.9718x; 14.9718x over previous
import functools

import jax
import jax.numpy as jnp
from jax import lax
from jax.experimental import pallas as pl
from jax.experimental.pallas import tpu as pltpu
from jax.experimental.pallas import tpu_sc as plsc

OUT_H = 7
OUT_W = 7
SPATIAL_SCALE = 0.125
SAMPLING_RATIO = 2

L = 16


def _full(v, dtype=jnp.int32):
    return jnp.full((L,), v, dtype=dtype)


def _roi_align_sc(x_table, rois_pad, *, N, C, H, W, K, rois_per_worker):
    C32 = C // 32
    NSTEP = rois_per_worker * OUT_H

    mesh = plsc.VectorSubcoreMesh(core_axis_name="c", subcore_axis_name="s")

    @functools.partial(
        pl.kernel,
        mesh=mesh,
        compiler_params=pltpu.CompilerParams(needs_layout_passes=False),
        out_type=jax.ShapeDtypeStruct((K, OUT_H * OUT_W * C), jnp.float32),
        scratch_types=[
            pltpu.VMEM((rois_per_worker, L), jnp.float32),
            pltpu.VMEM((L,), jnp.float32),
            pltpu.VMEM((2, L), jnp.float32),
            pltpu.VMEM((2, L), jnp.int32),
            pltpu.VMEM((2, 8, L), jnp.float32),
            pltpu.VMEM((2, 8 * 14), jnp.int32),
            pltpu.VMEM((2, 8 * 14, C // 2), jnp.int32),
            pltpu.VMEM((2, OUT_H * OUT_W * C), jnp.float32),
            pltpu.SemaphoreType.DMA((2,)),
            pltpu.SemaphoreType.DMA((2,)),
        ],
    )
    def k(table, rois, out, rois_all, roi_v, ay_v, yb_v, wb_v, idx_v, rows_v,
          acc_v, sems, osems):
        wid = lax.axis_index("s") * 2 + lax.axis_index("c")
        pltpu.sync_copy(rois.at[pl.ds(wid * rois_per_worker, rois_per_worker)],
                        rois_all)

        iota = jnp.arange(L, dtype=jnp.int32)
        iotaf = iota.astype(jnp.float32)
        t = iotaf * 0.5 + 0.25

        def issue(s, slot):
            roi = s // OUT_H
            ph = s % OUT_H
            rv = plsc.load_gather(rois_all, [_full(roi), iota])
            roi_v[...] = rv

            def bc(lane):
                return plsc.load_gather(roi_v, [_full(lane)])

            b = bc(0).astype(jnp.int32)
            sy1 = bc(1) * SPATIAL_SCALE
            sx1 = bc(2) * SPATIAL_SCALE
            sy2 = bc(3) * SPATIAL_SCALE
            sx2 = bc(4) * SPATIAL_SCALE
            bin_h = jnp.maximum(sy2 - sy1, 1.0) * (1.0 / OUT_H)
            bin_w = jnp.maximum(sx2 - sx1, 1.0) * (1.0 / OUT_W)

            xs = sx1 + t * bin_w
            vx = jnp.where((xs >= -1.0) & (xs <= float(W)), 1.0, 0.0)
            xc = jnp.clip(xs, 0.0, float(W - 1))
            x0 = xc.astype(jnp.int32)
            fx = xc - x0.astype(jnp.float32)
            x1i = jnp.minimum(x0 + 1, W - 1)
            ax0 = (1.0 - fx) * vx
            ax1 = fx * vx

            ys = sy1 + t * bin_h
            vy = jnp.where((ys >= -1.0) & (ys <= float(H)), 0.25, 0.0)
            yc = jnp.clip(ys, 0.0, float(H - 1))
            y0 = yc.astype(jnp.int32)
            fy = yc - y0.astype(jnp.float32)
            y1i = jnp.minimum(y0 + 1, H - 1)
            bbase = b * (H * W)
            yb_v[0] = bbase + y0 * W
            yb_v[1] = bbase + y1i * W
            ay_v[0] = (1.0 - fy) * vy
            ay_v[1] = fy * vy

            msk = iota < 14
            for iy in range(2):
                sy = 2 * ph + iy
                yb0 = plsc.load_gather(yb_v, [_full(0), _full(sy)])
                yb1 = plsc.load_gather(yb_v, [_full(1), _full(sy)])
                ay0 = plsc.load_gather(ay_v, [_full(0), _full(sy)])
                ay1 = plsc.load_gather(ay_v, [_full(1), _full(sy)])
                g = iy * 4
                for gg, vals in (
                    (g + 0, yb0 + x0),
                    (g + 1, yb0 + x1i),
                    (g + 2, yb1 + x0),
                    (g + 3, yb1 + x1i),
                ):
                    plsc.store_scatter(
                        idx_v, [_full(slot), iota + gg * 14], vals, mask=msk)
                wb_v[slot, g + 0] = ay0 * ax0
                wb_v[slot, g + 1] = ay0 * ax1
                wb_v[slot, g + 2] = ay1 * ax0
                wb_v[slot, g + 3] = ay1 * ax1

            pltpu.async_copy(table.at[idx_v.at[slot]], rows_v.at[slot],
                             sems.at[slot])

        def compute(s, slot):
            roi = s // OUT_H
            ph = s % OUT_H
            aslot = roi % 2
            rbase = wid * rois_per_worker
            pltpu.make_async_copy(table.at[idx_v.at[slot]], rows_v.at[slot],
                                  sems.at[slot]).wait()

            @pl.when(jnp.logical_and(ph == 0, roi >= 2))
            def _():
                pltpu.make_async_copy(acc_v.at[aslot], out.at[rbase + roi - 2],
                                      osems.at[aslot]).wait()

            for pw in range(OUT_W):
                w = [
                    plsc.load_gather(
                        wb_v, [_full(slot), _full(g), _full(2 * pw + jj)])
                    for g in range(8)
                    for jj in range(2)
                ]
                wp = [
                    plsc.pack(wi, wi, format=plsc.PackFormat.INTERLEAVED)
                    for wi in w
                ]
                base = (ph * OUT_W + pw) * C

                def row32(row, ch):
                    return plsc.bitcast(
                        rows_v[slot, row, pl.ds(ch * L, L)], jnp.bfloat16)

                for ch in range(C32):
                    acc = wp[0] * row32(2 * pw, ch)
                    for g in range(8):
                        for jj in range(2):
                            if g == 0 and jj == 0:
                                continue
                            acc = acc + wp[g * 2 + jj] * row32(
                                g * 14 + 2 * pw + jj, ch)
                    alo, ahi = plsc.unpack(
                        acc, format=plsc.PackFormat.INTERLEAVED)
                    acc_v[aslot, pl.ds(base + ch * 32, L)] = alo
                    acc_v[aslot, pl.ds(base + ch * 32 + L, L)] = ahi

            @pl.when(ph == OUT_H - 1)
            def _():
                pltpu.async_copy(acc_v.at[aslot], out.at[rbase + roi],
                                 osems.at[aslot])

        issue(0, 0)

        def pair_body(p, carry):
            s0 = 2 * p
            s1 = s0 + 1
            issue(s1, 1)
            compute(s0, 0)

            @pl.when(s1 + 1 < NSTEP)
            def _():
                issue(s1 + 1, 0)

            compute(s1, 1)
            return carry

        lax.fori_loop(0, NSTEP // 2, pair_body, 0)

        for a in range(2):
            rloc = rois_per_worker - 2 + a
            pltpu.make_async_copy(acc_v.at[rloc % 2],
                                  out.at[wid * rois_per_worker + rloc],
                                  osems.at[rloc % 2]).wait()

    return k(x_table, rois_pad)


def kernel(input, rois):
    N, C, H, W = input.shape
    K = rois.shape[0]
    x_table = jnp.transpose(input, (0, 2, 3, 1)).reshape(N * H * W, C)
    xb = x_table.astype(jnp.bfloat16).reshape(N * H * W, C // 32, 32)
    xb = jnp.stack([xb[:, :, :16], xb[:, :, 16:]], axis=-1)
    x_swz = lax.bitcast_convert_type(
        xb.reshape(N * H * W, C // 2, 2), jnp.int32)
    rois_pad = jnp.pad(rois, ((0, 0), (0, L - rois.shape[1])))
    out = _roi_align_sc(
        x_swz, rois_pad, N=N, C=C, H=H, W=W, K=K, rois_per_worker=K // 32
    )
    out = out.reshape(K, OUT_H, OUT_W, C)
    return jnp.transpose(out, (0, 3, 1, 2))
